# baseline (speedup 1.0000x reference)
import functools

import numpy as np

import jax
import jax.numpy as jnp
from jax import lax
from jax.experimental import pallas as pl
from jax.experimental.pallas import tpu as pltpu
from jax.experimental.pallas import tpu_sc as plsc

N = 2048
K = 128
BM = 1024
BN = 1024

NW = 32
ROWS_W = N // NW
CAP_T = 256
CAND = NW * CAP_T
CSEL = 256
IMAX = np.int32(0x7FFFFFFF)
IMIN = np.int32(-0x80000000)


def _grad_mask_kernel(w_ref, a_ref, out_ref, rmin_ref):
    p = pl.program_id(0)
    i = (p + 1) // 2
    j = p // 2

    d1 = lax.dot_general(
        w_ref[pl.ds(i * BM, BM), :], a_ref[pl.ds(j * BN, BN), :],
        (((1,), (1,)), ((), ())), preferred_element_type=jnp.float32)
    d2 = lax.dot_general(
        a_ref[:, pl.ds(i * BM, BM)], w_ref[:, pl.ds(j * BN, BN)],
        (((0,), (0,)), ((), ())), preferred_element_type=jnp.float32)

    rows = i * BM + lax.broadcasted_iota(jnp.int32, (BM, BN), 0)
    cols = j * BN + lax.broadcasted_iota(jnp.int32, (BM, BN), 1)
    g = jnp.where(cols < rows, d1 + d2, jnp.inf)
    u = lax.bitcast_convert_type(g, jnp.int32)
    skey = u ^ ((u >> 31) & IMAX)
    out_ref[...] = skey
    bmin = jnp.min(skey, axis=1, keepdims=True)

    @pl.when(j == 0)
    def _init():
        rmin_ref[pl.ds(i * BM, BM), :] = bmin

    @pl.when(j > 0)
    def _acc():
        rmin_ref[pl.ds(i * BM, BM), :] = jnp.minimum(
            rmin_ref[pl.ds(i * BM, BM), :], bmin)


def _masked_gradient_keys(adj, W):
    return pl.pallas_call(
        _grad_mask_kernel,
        grid=(3,),
        in_specs=[
            pl.BlockSpec((N, N), lambda p: (0, 0)),
            pl.BlockSpec((N, N), lambda p: (0, 0)),
        ],
        out_specs=(
            pl.BlockSpec((BM, BN), lambda p: ((p + 1) // 2, p // 2)),
            pl.BlockSpec((N, 1), lambda p: (0, 0)),
        ),
        out_shape=(jax.ShapeDtypeStruct((N, N), jnp.int32),
                   jax.ShapeDtypeStruct((N, 1), jnp.int32)),
    )(W, adj)


def _worker_id():
    return lax.axis_index("s") * 2 + lax.axis_index("c")


def _extract_sc(keys, rmin):
    mesh = plsc.VectorSubcoreMesh(core_axis_name="c", subcore_axis_name="s")

    @functools.partial(
        pl.kernel,
        mesh=mesh,
        compiler_params=pltpu.CompilerParams(needs_layout_passes=False),
        out_type=(jax.ShapeDtypeStruct((CAND,), jnp.int32),
                  jax.ShapeDtypeStruct((CAND,), jnp.int32)),
        scratch_types=[
            pltpu.VMEM((N,), jnp.int32),
            pltpu.VMEM((N,), jnp.int32),
            pltpu.VMEM((CAP_T + 16,), jnp.int32),
            pltpu.VMEM((CAP_T + 16,), jnp.int32),
        ],
    )
    def ext_kernel(k_hbm, rm_hbm, out_v, out_i, rmv, rowbuf, cv, ci):
        wid = _worker_id()
        lanes = lax.iota(jnp.int32, 16)
        zero16 = jnp.zeros((16,), jnp.int32)
        imax16 = jnp.full((16,), IMAX, jnp.int32)

        pltpu.sync_copy(rm_hbm, rmv)

        def bs_body(_, carry):
            lo, hi = carry
            mid = lo + lax.shift_right_logical(hi - lo, 1)

            def cnt(b, acc):
                rv = rmv[pl.ds(b * 16, 16)]
                return acc + jnp.where(rv <= mid, 1, 0).astype(jnp.int32)
            acc = lax.fori_loop(0, N // 16, cnt, zero16, unroll=8)
            big = jnp.sum(acc) >= K
            return (jnp.where(big, lo, mid), jnp.where(big, mid, hi))
        _, thresh = lax.fori_loop(0, 32, bs_body, (IMIN, IMAX))

        def c_init(b, _):
            cv[pl.ds(b * 16, 16)] = imax16
            ci[pl.ds(b * 16, 16)] = zero16
            return 0
        lax.fori_loop(0, (CAP_T + 16) // 16, c_init, 0)

        def scan_row(row, ptr):
            pltpu.sync_copy(k_hbm.at[row], rowbuf)

            def v_body(i, ptr):
                sk = rowbuf[pl.ds(i * 16, 16)]
                colv = i * 16 + lanes
                m = (sk <= thresh) & (colv < row)
                idxv = row * N + colv
                p = jnp.minimum(ptr, CAP_T)
                plsc.store_compressed(cv.at[pl.ds(p, 16)], sk, mask=m)
                plsc.store_compressed(ci.at[pl.ds(p, 16)], idxv, mask=m)
                cnt16 = plsc.all_reduce_population_count(m)
                return ptr + cnt16[0]
            return lax.fori_loop(0, (row + 15) >> 4, v_body, ptr)

        def group_body(g, ptr):
            rv = rmv[pl.ds(wid * ROWS_W + g * 16, 16)]
            for l in range(16):
                row = wid * ROWS_W + g * 16 + l
                ptr = lax.cond(rv[l] <= thresh,
                               functools.partial(scan_row, row),
                               lambda p: p, ptr)
            return ptr
        lax.fori_loop(0, ROWS_W // 16, group_body, jnp.int32(0))

        pltpu.sync_copy(cv.at[pl.ds(0, CAP_T)],
                        out_v.at[pl.ds(wid * CAP_T, CAP_T)])
        pltpu.sync_copy(ci.at[pl.ds(0, CAP_T)],
                        out_i.at[pl.ds(wid * CAP_T, CAP_T)])

    return ext_kernel(keys, rmin)


def _final_kernel(v_ref, i_ref, row_ref, col_ref):
    v = v_ref[...]
    ii = i_ref[...]
    lane = lax.broadcasted_iota(jnp.int32, (1, K), 1)

    def body(t, carry):
        v, rows, cols = carry
        m = jnp.min(v)
        am = jnp.min(jnp.where(v == m, ii, IMAX))
        pred = lane == t
        rows = jnp.where(pred, am >> 11, rows)
        cols = jnp.where(pred, am & (N - 1), cols)
        v = jnp.where(ii == am, IMAX, v)
        return (v, rows, cols)

    zero = jnp.zeros((1, K), jnp.int32)
    _, rows, cols = lax.fori_loop(0, K, body, (v, zero, zero))
    row_ref[...] = rows
    col_ref[...] = cols


def _final_topk(cv, ci):
    return pl.pallas_call(
        _final_kernel,
        out_shape=(jax.ShapeDtypeStruct((1, K), jnp.int32),
                   jax.ShapeDtypeStruct((1, K), jnp.int32)),
    )(cv.reshape(K, CAND // K), ci.reshape(K, CAND // K))


def kernel(adj, W):
    keys, rmin = _masked_gradient_keys(adj, W)
    cv, ci = _extract_sc(keys, rmin.reshape(-1))
    rows, cols = _final_topk(cv, ci)
    actions = jnp.stack([rows[0], cols[0]], axis=-1)
    return (actions, jnp.zeros((1,), dtype=jnp.float32))
